# baseline (speedup 1.0000x reference)
import functools

import jax
import jax.numpy as jnp
from jax import lax
from jax.experimental import pallas as pl
from jax.experimental.pallas import tpu as pltpu
from jax.experimental.pallas import tpu_sc as plsc

SEQ = 8192
D = 2048
LANES = 16
NC = 2
NS = 16
NW = NC * NS

SC_ROWS = 4096
TC_ROWS = SEQ - SC_ROWS
ROWS_PER_W = SC_ROWS // NW
CHUNK = 8
NSTEPS = ROWS_PER_W // CHUNK
NPAIRS = NSTEPS // 2
STRIPS = D // LANES

BT = 256
TC_OFF = SC_ROWS // BT


def _sc_body(x_hbm, w_hbm, pos_hbm, out_hbm,
             idx_v,
             x0, g0, o0, x1, g1, o1,
             gs0, xs0, os0, gs1, xs1, os1):
    wid = lax.axis_index("s") * NC + lax.axis_index("c")
    base = wid * ROWS_PER_W

    pltpu.sync_copy(pos_hbm.at[pl.ds(base, ROWS_PER_W)], idx_v)

    def start_loads(s, x_v, g_v, gsem, xsem):
        row0 = base + s * CHUNK
        pltpu.async_copy(w_hbm.at[idx_v.at[pl.ds(s * CHUNK, CHUNK)]], g_v, gsem)
        pltpu.async_copy(x_hbm.at[pl.ds(row0, CHUNK)], x_v, xsem)

    def wait_loads(s, x_v, g_v, gsem, xsem):
        pltpu.make_async_copy(w_hbm.at[idx_v.at[pl.ds(s * CHUNK, CHUNK)]],
                              g_v, gsem).wait()
        pltpu.make_async_copy(x_hbm.at[pl.ds(base, CHUNK)], x_v, xsem).wait()

    def add_chunk(x_v, g_v, o_v):
        def row_body(r, c2):
            for c in range(STRIPS):
                sl = pl.ds(c * LANES, LANES)
                o_v[r, sl] = x_v[r, sl] + g_v[r, sl]
            return c2
        lax.fori_loop(0, CHUNK, row_body, 0, unroll=False)

    def start_store(s, o_v, osem):
        row0 = base + s * CHUNK
        pltpu.async_copy(o_v, out_hbm.at[pl.ds(row0, CHUNK)], osem)

    def wait_store(o_v, osem):
        pltpu.make_async_copy(o_v, out_hbm.at[pl.ds(base, CHUNK)], osem).wait()

    start_loads(0, x0, g0, gs0, xs0)
    start_loads(1, x1, g1, gs1, xs1)

    def pair(p, carry):
        s0 = 2 * p
        s1 = s0 + 1

        wait_loads(s0, x0, g0, gs0, xs0)

        @pl.when(p > 0)
        def _():
            wait_store(o0, os0)

        add_chunk(x0, g0, o0)
        start_store(s0, o0, os0)

        @pl.when(p < NPAIRS - 1)
        def _():
            start_loads(s0 + 2, x0, g0, gs0, xs0)

        wait_loads(s1, x1, g1, gs1, xs1)

        @pl.when(p > 0)
        def _():
            wait_store(o1, os1)

        add_chunk(x1, g1, o1)
        start_store(s1, o1, os1)

        @pl.when(p < NPAIRS - 1)
        def _():
            start_loads(s1 + 2, x1, g1, gs1, xs1)

        return carry

    lax.fori_loop(0, NPAIRS, pair, 0)

    wait_store(o0, os0)
    wait_store(o1, os1)


def _sc_part(x, W, pos):
    mesh = plsc.VectorSubcoreMesh(core_axis_name="c", subcore_axis_name="s")
    f = pl.kernel(
        _sc_body,
        mesh=mesh,
        out_type=jax.ShapeDtypeStruct((SC_ROWS, D), jnp.float32),
        scratch_types=[
            pltpu.VMEM((ROWS_PER_W,), jnp.int32),
            pltpu.VMEM((CHUNK, D), jnp.float32),
            pltpu.VMEM((CHUNK, D), jnp.float32),
            pltpu.VMEM((CHUNK, D), jnp.float32),
            pltpu.VMEM((CHUNK, D), jnp.float32),
            pltpu.VMEM((CHUNK, D), jnp.float32),
            pltpu.VMEM((CHUNK, D), jnp.float32),
            pltpu.SemaphoreType.DMA,
            pltpu.SemaphoreType.DMA,
            pltpu.SemaphoreType.DMA,
            pltpu.SemaphoreType.DMA,
            pltpu.SemaphoreType.DMA,
            pltpu.SemaphoreType.DMA,
        ],
    )
    return f(x, W, pos)


def _tc_add(pos_ref, x_ref, w_ref, o_ref):
    o_ref[...] = x_ref[...] + w_ref[...]


def _tc_part(x, W, pos):
    grid_spec = pltpu.PrefetchScalarGridSpec(
        num_scalar_prefetch=1,
        grid=(TC_ROWS // BT,),
        in_specs=[
            pl.BlockSpec((BT, D), lambda i, pos_ref: (TC_OFF + i, 0)),
            pl.BlockSpec((BT, D),
                         lambda i, pos_ref: (pos_ref[(TC_OFF + i) * BT] // BT, 0)),
        ],
        out_specs=pl.BlockSpec((BT, D), lambda i, pos_ref: (i, 0)),
    )
    return pl.pallas_call(
        _tc_add,
        grid_spec=grid_spec,
        out_shape=jax.ShapeDtypeStruct((TC_ROWS, D), jnp.float32),
    )(pos, x, W)


@jax.jit
def kernel(x, W, pos):
    sc_out = _sc_part(x, W, pos)
    tc_out = _tc_part(x, W, pos)
    return jnp.concatenate([sc_out, tc_out], axis=0)
